# Initial kernel scaffold; baseline (speedup 1.0000x reference)
import jax
import jax.numpy as jnp
from jax.experimental import pallas as pl


def kernel(ts_feat, adj_indices, adj_values, label_indices, W1, gamma1, beta1, W2, gamma2, beta2, Wc1, bc1, Wc2, bc2):
    raise NotImplementedError("write your pallas kernel here")


# speedup vs baseline: 4.4070x; 4.4070x over previous
import functools

import jax
import jax.numpy as jnp
from jax import lax
from jax.experimental import pallas as pl
from jax.experimental.pallas import tpu as pltpu
from jax.experimental.pallas import tpu_sc as plsc

N = 10000
E = 320000
D = 128
BN_EPS = 1e-3

NC = 2
NS = 16
NW = NC * NS

EPW = E // NW
K = 80
NCHUNK = EPW // K
NPAD = 10240
RPT = NPAD // NS

_LANES = 16
_FPV = D // _LANES


def _iota16(v):
    return jnp.full((_LANES,), v, dtype=jnp.int32)


@functools.partial(
    pl.kernel,
    out_type=jax.ShapeDtypeStruct((NC, NPAD, D), jnp.float32),
    mesh=plsc.VectorSubcoreMesh(core_axis_name="c", subcore_axis_name="s"),
    scratch_types=[
        pltpu.VMEM((2, K), jnp.int32),
        pltpu.VMEM((K,), jnp.float32),
        pltpu.VMEM((K, D), jnp.float32),
        pltpu.VMEM_SHARED((NPAD, D), jnp.float32),
        pltpu.SemaphoreType.DMA,
    ],
)
def _sc_aggregate(x_hbm, sd_hbm, val_hbm, out_hbm,
                  sd, vals, rows, agg_sh, sem):
    c = lax.axis_index("c")
    s = lax.axis_index("s")
    wid = c * NS + s

    zero = jnp.zeros((_LANES,), jnp.float32)
    def zrow(r, carry):
        for f in range(_FPV):
            rows[r, pl.ds(f * _LANES, _LANES)] = zero
        return carry
    lax.fori_loop(0, K, zrow, 0)
    base = s * RPT
    for t in range(RPT // K):
        pltpu.sync_copy(rows, agg_sh.at[pl.ds(base + t * K, K)])
    plsc.subcore_barrier()

    def chunk(j, carry):
        pltpu.sync_copy(sd_hbm.at[wid, j], sd)
        pltpu.sync_copy(val_hbm.at[wid, j], vals)
        pltpu.async_copy(x_hbm.at[sd.at[0]], rows, sem).wait()
        def group(g, carry2):
            gbase = g * _LANES
            vals16 = vals[pl.ds(gbase, _LANES)]
            for e in range(_LANES):
                r = gbase + e
                w = vals16[e]
                for f in range(_FPV):
                    sl = pl.ds(f * _LANES, _LANES)
                    rows[r, sl] = rows[r, sl] * w
            return carry2

        lax.fori_loop(0, K // _LANES, group, 0)
        pltpu.sync_copy(rows, agg_sh.at[sd.at[1]], add=True)
        return carry

    lax.fori_loop(0, NCHUNK, chunk, 0)
    plsc.subcore_barrier()

    pltpu.sync_copy(agg_sh.at[pl.ds(base, RPT)],
                    out_hbm.at[c, pl.ds(base, RPT)])


BPAD = 10240
BPW = BPAD // NW


@functools.partial(
    pl.kernel,
    out_type=jax.ShapeDtypeStruct((BPAD, D), jnp.float32),
    mesh=plsc.VectorSubcoreMesh(core_axis_name="c", subcore_axis_name="s"),
    scratch_types=[
        pltpu.VMEM((BPW,), jnp.int32),
        pltpu.VMEM((BPW, D), jnp.float32),
        pltpu.SemaphoreType.DMA,
    ],
)
def _sc_gather_rows(table_hbm, idx_hbm, out_hbm, idx_v, rows_v, sem):
    wid = lax.axis_index("c") * NS + lax.axis_index("s")
    base = wid * BPW
    pltpu.sync_copy(idx_hbm.at[pl.ds(base, BPW)], idx_v)
    pltpu.async_copy(table_hbm.at[idx_v], rows_v, sem).wait()
    pltpu.sync_copy(rows_v, out_hbm.at[pl.ds(base, BPW)])


def _dense_body(p_ref, w_ref, g_ref, b_ref, xs_ref, o_ref):
    agg = p_ref[0] + p_ref[1]
    h = jnp.dot(agg, w_ref[...], preferred_element_type=jnp.float32)
    mean = jnp.sum(h, axis=0, keepdims=True) / N
    var = jnp.sum(h * h, axis=0, keepdims=True) / N - mean * mean
    hn = (h[:N] - mean) * lax.rsqrt(var + BN_EPS) * g_ref[...] + b_ref[...]
    o_ref[...] = jnp.maximum(hn + xs_ref[...], 0.0)


def _tc_dense(p, w, gamma, beta, xskip):
    return pl.pallas_call(
        _dense_body,
        out_shape=jax.ShapeDtypeStruct((N, D), jnp.float32),
    )(p, w, gamma.reshape(1, D), beta.reshape(1, D), xskip)


def _cls_body(g_ref, w1_ref, b1_ref, w2_ref, b2_ref, o_ref):
    hid = jnp.dot(g_ref[...], w1_ref[...], preferred_element_type=jnp.float32)
    hid = jnp.maximum(hid + b1_ref[...], 0.0)
    o_ref[...] = jnp.dot(hid, w2_ref[...],
                         preferred_element_type=jnp.float32) + b2_ref[...]


def _tc_classifier(g, w1, b1, w2, b2):
    return pl.pallas_call(
        _cls_body,
        out_shape=jax.ShapeDtypeStruct((BPAD, 2), jnp.float32),
    )(g, w1, b1.reshape(1, -1), w2, b2.reshape(1, -1))


def kernel(ts_feat, adj_indices, adj_values, label_indices,
           W1, gamma1, beta1, W2, gamma2, beta2,
           Wc1, bc1, Wc2, bc2):
    sd = jnp.stack(
        [adj_indices[1].reshape(NW, NCHUNK, K),
         adj_indices[0].reshape(NW, NCHUNK, K)], axis=2)
    val = adj_values.reshape(NW, NCHUNK, K)

    p1 = _sc_aggregate(ts_feat, sd, val)
    h1 = _tc_dense(p1, W1, gamma1, beta1, ts_feat)
    p2 = _sc_aggregate(h1, sd, val)
    h2 = _tc_dense(p2, W2, gamma2, beta2, h1)

    labels_pad = jnp.concatenate(
        [label_indices, jnp.zeros((BPAD - N,), jnp.int32)])
    gathered = _sc_gather_rows(h2, labels_pad)
    logits = _tc_classifier(gathered, Wc1, bc1, Wc2, bc2)
    return logits[:N]


# speedup vs baseline: 9.8936x; 2.2450x over previous
import functools

import jax
import jax.numpy as jnp
from jax import lax
from jax.experimental import pallas as pl
from jax.experimental.pallas import tpu as pltpu
from jax.experimental.pallas import tpu_sc as plsc

N = 10000
E = 320000
D = 128
BN_EPS = 1e-3

NC = 2
NS = 16
NW = NC * NS

EPW = E // NW
K = 80
NCHUNK = EPW // K
NPAD = 10240
RPT = NPAD // NS

_LANES = 16
_FPV = D // _LANES


def _iota16(v):
    return jnp.full((_LANES,), v, dtype=jnp.int32)


NBUF = 4
NQUAD = 31
_CBYTES = K * D * 4


@functools.partial(
    pl.kernel,
    out_type=jax.ShapeDtypeStruct((NC, NPAD, D), jnp.float32),
    mesh=plsc.VectorSubcoreMesh(core_axis_name="c", subcore_axis_name="s"),
    scratch_types=[
        pltpu.VMEM((2, 4, 2, K), jnp.int32),
        pltpu.VMEM((2, 4, 1, K), jnp.float32),
        pltpu.VMEM((NBUF, K, D), jnp.float32),
        pltpu.VMEM_SHARED((NPAD, D), jnp.float32),
        [pltpu.SemaphoreType.DMA] * NBUF,
        [pltpu.SemaphoreType.DMA] * NBUF,
        [pltpu.SemaphoreType.DMA] * 2,
    ],
)
def _sc_aggregate(x_hbm, sd_hbm, val_hbm, out_hbm,
                  sdq, valq, rows, agg_sh, gsem, ssem, stsem):
    c = lax.axis_index("c")
    s = lax.axis_index("s")
    wid = c * NS + s

    def scale_rows(u, qb, sl):
        def group(g, carry):
            gbase = g * _LANES
            vals16 = valq[qb, sl, 0, pl.ds(gbase, _LANES)]
            for e in range(_LANES):
                r = gbase + e
                w = vals16[e]
                for f in range(_FPV):
                    csl = pl.ds(f * _LANES, _LANES)
                    rows[u, r, csl] = rows[u, r, csl] * w
            return carry
        lax.fori_loop(0, K // _LANES, group, 0)

    def issue_stage(q, qb):
        for i in range(4):
            pltpu.async_copy(sd_hbm.at[wid, q * 4 + i], sdq.at[qb, i],
                             stsem[qb])
            pltpu.async_copy(val_hbm.at[wid, q * 4 + i], valq.at[qb, i],
                             stsem[qb])

    def wait_stage(qb):
        for i in range(4):
            pltpu.make_async_copy(sd_hbm.at[wid, 0], sdq.at[qb, i],
                                  stsem[qb]).wait()
            pltpu.make_async_copy(val_hbm.at[wid, 0], valq.at[qb, i],
                                  stsem[qb]).wait()

    def issue_gather(u, qb, sl):
        pltpu.async_copy(x_hbm.at[sdq.at[qb, sl, 0]], rows.at[u], gsem[u])

    def wait_gather(u):
        pltpu.make_async_copy(x_hbm.at[pl.ds(0, K)], rows.at[u],
                              gsem[u]).wait()

    def issue_scatter(u, qb, sl):
        pltpu.async_copy(rows.at[u], agg_sh.at[sdq.at[qb, sl, 1]], ssem[u],
                         add=True)

    def wait_scatter(u):
        pltpu.make_async_copy(rows.at[u], agg_sh.at[pl.ds(0, K)],
                              ssem[u]).wait()

    zero = jnp.zeros((_LANES,), jnp.float32)
    def zrow(r, carry):
        for f in range(_FPV):
            rows[0, r, pl.ds(f * _LANES, _LANES)] = zero
        return carry
    lax.fori_loop(0, K, zrow, 0)
    base = s * RPT
    for t in range(RPT // K):
        pltpu.sync_copy(rows.at[0], agg_sh.at[pl.ds(base + t * K, K)])
    plsc.subcore_barrier()

    issue_stage(0, 0)
    wait_stage(0)
    issue_stage(1, 1)
    issue_gather(0, 0, 0)
    issue_gather(1, 0, 1)

    def quad(q, qb, last):
        qn = 1 - qb
        for u in range(4):
            if u == 2 and not last:
                @pl.when(q < NQUAD - 1)
                def _():
                    wait_stage(qn)
            wait_gather(u)
            scale_rows(u, qb, u)
            issue_scatter(u, qb, u)
            u2 = (u + 2) % 4
            if u < 2:
                @pl.when(q > 0)
                def _():
                    wait_scatter(u2)
                issue_gather(u2, qb, u + 2)
            else:
                wait_scatter(u2)
                if not last:
                    @pl.when(q < NQUAD - 1)
                    def _():
                        issue_gather(u2, qn, u - 2)
        if not last:
            @pl.when(q < NQUAD - 2)
            def _():
                issue_stage(q + 2, qb)

    def pair(t, carry):
        q0 = 2 * t
        quad(q0, 0, False)
        quad(q0 + 1, 1, False)
        return carry

    lax.fori_loop(0, NQUAD // 2, pair, 0)
    quad(NQUAD - 1, (NQUAD - 1) % 2, True)

    wait_scatter(2)
    wait_scatter(3)

    pltpu.sync_copy(sd_hbm.at[wid, NCHUNK - 1], sdq.at[0, 0])
    pltpu.sync_copy(val_hbm.at[wid, NCHUNK - 1], valq.at[0, 0])
    issue_gather(0, 0, 0)
    wait_gather(0)
    scale_rows(0, 0, 0)
    issue_scatter(0, 0, 0)
    wait_scatter(0)

    plsc.subcore_barrier()

    pltpu.sync_copy(agg_sh.at[pl.ds(base, RPT)],
                    out_hbm.at[c, pl.ds(base, RPT)])


BPAD = 10240
BPW = BPAD // NW


@functools.partial(
    pl.kernel,
    out_type=jax.ShapeDtypeStruct((BPAD, D), jnp.float32),
    mesh=plsc.VectorSubcoreMesh(core_axis_name="c", subcore_axis_name="s"),
    scratch_types=[
        pltpu.VMEM((BPW,), jnp.int32),
        pltpu.VMEM((BPW, D), jnp.float32),
        pltpu.SemaphoreType.DMA,
    ],
)
def _sc_gather_rows(table_hbm, idx_hbm, out_hbm, idx_v, rows_v, sem):
    wid = lax.axis_index("c") * NS + lax.axis_index("s")
    base = wid * BPW
    pltpu.sync_copy(idx_hbm.at[pl.ds(base, BPW)], idx_v)
    pltpu.async_copy(table_hbm.at[idx_v], rows_v, sem).wait()
    pltpu.sync_copy(rows_v, out_hbm.at[pl.ds(base, BPW)])


def _dense_body(p_ref, w_ref, g_ref, b_ref, xs_ref, o_ref):
    agg = p_ref[0] + p_ref[1]
    h = jnp.dot(agg, w_ref[...], preferred_element_type=jnp.float32)
    mean = jnp.sum(h, axis=0, keepdims=True) / N
    var = jnp.sum(h * h, axis=0, keepdims=True) / N - mean * mean
    hn = (h[:N] - mean) * lax.rsqrt(var + BN_EPS) * g_ref[...] + b_ref[...]
    o_ref[...] = jnp.maximum(hn + xs_ref[...], 0.0)


def _tc_dense(p, w, gamma, beta, xskip):
    return pl.pallas_call(
        _dense_body,
        out_shape=jax.ShapeDtypeStruct((N, D), jnp.float32),
    )(p, w, gamma.reshape(1, D), beta.reshape(1, D), xskip)


def _cls_body(g_ref, w1_ref, b1_ref, w2_ref, b2_ref, o_ref):
    hid = jnp.dot(g_ref[...], w1_ref[...], preferred_element_type=jnp.float32)
    hid = jnp.maximum(hid + b1_ref[...], 0.0)
    o_ref[...] = jnp.dot(hid, w2_ref[...],
                         preferred_element_type=jnp.float32) + b2_ref[...]


def _tc_classifier(g, w1, b1, w2, b2):
    return pl.pallas_call(
        _cls_body,
        out_shape=jax.ShapeDtypeStruct((BPAD, 2), jnp.float32),
    )(g, w1, b1.reshape(1, -1), w2, b2.reshape(1, -1))


def kernel(ts_feat, adj_indices, adj_values, label_indices,
           W1, gamma1, beta1, W2, gamma2, beta2,
           Wc1, bc1, Wc2, bc2):
    sd = jnp.stack(
        [adj_indices[1].reshape(NW, NCHUNK, K),
         adj_indices[0].reshape(NW, NCHUNK, K)], axis=2)
    val = adj_values.reshape(NW, NCHUNK, 1, K)

    p1 = _sc_aggregate(ts_feat, sd, val)
    h1 = _tc_dense(p1, W1, gamma1, beta1, ts_feat)
    p2 = _sc_aggregate(h1, sd, val)
    h2 = _tc_dense(p2, W2, gamma2, beta2, h1)

    labels_pad = jnp.concatenate(
        [label_indices, jnp.zeros((BPAD - N,), jnp.int32)])
    gathered = _sc_gather_rows(h2, labels_pad)
    logits = _tc_classifier(gathered, Wc1, bc1, Wc2, bc2)
    return logits[:N]
